# Initial kernel scaffold; baseline (speedup 1.0000x reference)
import jax
import jax.numpy as jnp
from jax.experimental import pallas as pl


def kernel(x, lengths):
    raise NotImplementedError("write your pallas kernel here")


# speedup vs baseline: 8.5893x; 8.5893x over previous
import functools

import jax
import jax.numpy as jnp
from jax.experimental import pallas as pl
from jax.experimental.pallas import tpu as pltpu

_K = 3


def _body(csum_ref, halo_ref, x_ref, o_ref, *, rows_per_blk):
    i = pl.program_id(0)
    xb = x_ref[...]
    R, d = xb.shape

    prev_in = jnp.concatenate([halo_ref[0, 0:1, :], xb[:-1, :]], axis=0)
    next_in = jnp.concatenate([xb[1:, :], halo_ref[0, 1:2, :]], axis=0)

    g = i * rows_per_blk + jax.lax.broadcasted_iota(jnp.int32, (R, 1), 0)
    csum = csum_ref[...]
    is_start = jnp.any(g == csum, axis=1, keepdims=True) | (g == 0)
    is_end = jnp.any((g + 1) == csum, axis=1, keepdims=True)

    prev_m = jnp.where(is_start, 0.0, prev_in)
    next_m = jnp.where(is_end, 0.0, next_in)

    mod = jax.lax.broadcasted_iota(jnp.int32, (R, 3 * 128), 1) % _K
    idx = jax.lax.broadcasted_iota(jnp.int32, (R, 3 * 128), 1) // _K
    for a in range(d // 128):
        pa = jnp.take_along_axis(prev_m[:, 128 * a:128 * (a + 1)], idx, axis=1)
        ca = jnp.take_along_axis(xb[:, 128 * a:128 * (a + 1)], idx, axis=1)
        na = jnp.take_along_axis(next_m[:, 128 * a:128 * (a + 1)], idx, axis=1)
        o_ref[:, 384 * a:384 * (a + 1)] = jnp.where(
            mod == 0, pa, jnp.where(mod == 1, ca, na))


def kernel(x, lengths):
    N, d = x.shape
    lens = lengths.astype(jnp.int32)
    csum = jnp.cumsum(lens).reshape(1, -1)

    R = 640
    assert N % R == 0
    nblk = N // R

    blk = jnp.arange(nblk, dtype=jnp.int32)
    prev_idx = jnp.maximum(blk * R - 1, 0)
    next_idx = jnp.minimum((blk + 1) * R, N - 1)
    halo = jnp.stack([x[prev_idx], x[next_idx]], axis=1)

    out = pl.pallas_call(
        functools.partial(_body, rows_per_blk=R),
        grid=(nblk,),
        in_specs=[
            pl.BlockSpec((1, csum.shape[1]), lambda i: (0, 0)),
            pl.BlockSpec((1, 2, d), lambda i: (i, 0, 0)),
            pl.BlockSpec((R, d), lambda i: (i, 0)),
        ],
        out_specs=pl.BlockSpec((R, d * _K), lambda i: (i, 0)),
        out_shape=jax.ShapeDtypeStruct((N, d * _K), x.dtype),
    )(csum, halo, x)
    return out
